# Initial kernel scaffold; baseline (speedup 1.0000x reference)
import jax
import jax.numpy as jnp
from jax.experimental import pallas as pl


def kernel(x, neighbor_index, conv_w, conv_b, mlp_w, mlp_b):
    raise NotImplementedError("write your pallas kernel here")


# speedup vs baseline: 1.8877x; 1.8877x over previous
import functools

import jax
import jax.numpy as jnp
from jax import lax
from jax.experimental import pallas as pl
from jax.experimental.pallas import tpu as pltpu
from jax.experimental.pallas import tpu_sc as plsc

_CHUNK = 128


def _sc_gather(x2d, idx_pad, n_rows_pad):
  n, c = x2d.shape
  nw = 32
  per_w = n_rows_pad // nw
  nch = per_w // _CHUNK
  mesh = plsc.VectorSubcoreMesh(core_axis_name="c", subcore_axis_name="s")

  @functools.partial(
      pl.kernel,
      mesh=mesh,
      out_type=jax.ShapeDtypeStruct((n_rows_pad, c), jnp.float32),
      scratch_types=[
          pltpu.VMEM((_CHUNK,), jnp.int32),
          pltpu.VMEM((_CHUNK, c), jnp.float32),
          pltpu.SemaphoreType.DMA,
      ],
  )
  def gather_kernel(x_hbm, idx_hbm, out_hbm, idx_v, rows_v, sem):
    cid = lax.axis_index("c")
    sid = lax.axis_index("s")
    base = (sid * 2 + cid) * per_w

    def body(j, carry):
      off = base + j * _CHUNK
      pltpu.sync_copy(idx_hbm.at[pl.ds(off, _CHUNK)], idx_v)
      pltpu.async_copy(x_hbm.at[idx_v], rows_v, sem).wait()
      pltpu.sync_copy(rows_v, out_hbm.at[pl.ds(off, _CHUNK)])
      return carry

    lax.fori_loop(0, nch, body, 0)

  return gather_kernel(x2d, idx_pad)


def _dense_block(xg, mlp_wt, mlp_b, wt, conv_b, nb, k, c, oc):
  xg3 = xg.reshape(nb, k, c)
  rel = (xg3 - xg3[:, 0:1, :]).reshape(nb * k, c)
  logits = (
      jnp.dot(rel, mlp_wt, preferred_element_type=jnp.float32) + mlp_b
  )
  m = jnp.max(logits, axis=1, keepdims=True)
  e = jnp.exp(logits - m)
  p = e / jnp.sum(e, axis=1, keepdims=True)
  p3 = p.reshape(nb, k, k)

  acc = jnp.zeros((nb, k, c), jnp.float32)
  for k1 in range(k):
    acc = acc + p3[:, :, k1][:, :, None] * xg3[:, k1, :][:, None, :]

  out = jnp.broadcast_to(conv_b, (nb, oc))
  for k2 in range(k):
    out = out + jnp.dot(
        acc[:, k2, :], wt[k2], preferred_element_type=jnp.float32
    )
  return jnp.where(out > 0.0, out, jnp.exp(out) - 1.0)


def _tc_dense(xg2d, mlp_wt, mlp_b2, wt, conv_b2, n, nb, k, c, oc):
  nblk = n // nb

  def body(xg_ref, mlpwt_ref, mlpb_ref, wt_ref, convb_ref, out_ref):
    res = _dense_block(
        xg_ref[...], mlpwt_ref[...], mlpb_ref[...], wt_ref[...],
        convb_ref[...], nb, k, c, oc,
    )
    out_ref[...] = res

    @pl.when(pl.program_id(0) == nblk - 1)
    def _zero_last():
      out_ref[nb - 1 : nb, :] = jnp.zeros((1, oc), jnp.float32)

  return pl.pallas_call(
      body,
      grid=(nblk,),
      in_specs=[
          pl.BlockSpec((nb * k, c), lambda i: (i, 0)),
          pl.BlockSpec((c, k), lambda i: (0, 0)),
          pl.BlockSpec((1, k), lambda i: (0, 0)),
          pl.BlockSpec((k, c, oc), lambda i: (0, 0, 0)),
          pl.BlockSpec((1, oc), lambda i: (0, 0)),
      ],
      out_specs=pl.BlockSpec((nb, oc), lambda i: (i, 0)),
      out_shape=jax.ShapeDtypeStruct((n, oc), jnp.float32),
  )(xg2d, mlp_wt, mlp_b2, wt, conv_b2)


def kernel(x, neighbor_index, conv_w, conv_b, mlp_w, mlp_b):
  b, n, c = x.shape
  k = neighbor_index.shape[2]
  oc = conv_w.shape[0]
  assert b == 1

  x2d = x.reshape(n, c)
  ni = neighbor_index.reshape(n * k).astype(jnp.int32)

  rows = n * k
  rows_pad = ((rows + 32 * _CHUNK - 1) // (32 * _CHUNK)) * (32 * _CHUNK)
  ni_pad = jnp.concatenate(
      [ni, jnp.zeros((rows_pad - rows,), jnp.int32)]
  )

  xg2d = _sc_gather(x2d, ni_pad, rows_pad)

  wt = conv_w.reshape(oc, c, k).transpose(2, 1, 0)
  mlp_wt = mlp_w.T
  mlp_b2 = mlp_b.reshape(1, k)
  conv_b2 = conv_b.reshape(1, oc)

  nb = 400
  out2d = _tc_dense(xg2d, mlp_wt, mlp_b2, wt, conv_b2, n, nb, k, c, oc)
  return out2d.reshape(b, n, oc)


# speedup vs baseline: 2.0055x; 1.0624x over previous
import functools

import jax
import jax.numpy as jnp
from jax import lax
from jax.experimental import pallas as pl
from jax.experimental.pallas import tpu as pltpu
from jax.experimental.pallas import tpu_sc as plsc

_CHUNK = 128


def _sc_gather(x2d, idx_pad, n_rows_pad):
  n, c = x2d.shape
  nw = 32
  per_w = n_rows_pad // nw
  nch = per_w // _CHUNK
  mesh = plsc.VectorSubcoreMesh(core_axis_name="c", subcore_axis_name="s")

  assert nch % 2 == 0

  @functools.partial(
      pl.kernel,
      mesh=mesh,
      out_type=jax.ShapeDtypeStruct((n_rows_pad, c), jnp.float32),
      scratch_types=[
          pltpu.VMEM((per_w,), jnp.int32),
          pltpu.VMEM((_CHUNK, c), jnp.float32),
          pltpu.VMEM((_CHUNK, c), jnp.float32),
          pltpu.SemaphoreType.DMA,
          pltpu.SemaphoreType.DMA,
      ],
  )
  def gather_kernel(x_hbm, idx_hbm, out_hbm, idx_v, rows_a, rows_b, sem_a,
                    sem_b):
    cid = lax.axis_index("c")
    sid = lax.axis_index("s")
    base = (sid * 2 + cid) * per_w

    pltpu.sync_copy(idx_hbm.at[pl.ds(base, per_w)], idx_v)

    def gather(g, buf, sem):
      return pltpu.async_copy(
          x_hbm.at[idx_v.at[pl.ds(g * _CHUNK, _CHUNK)]], buf, sem
      )

    def writeback(g, buf):
      pltpu.sync_copy(buf, out_hbm.at[pl.ds(base + g * _CHUNK, _CHUNK)])

    gather(0, rows_a, sem_a)

    def pair(j, carry):
      g = j * 2
      gather(g + 1, rows_b, sem_b)
      pltpu.make_async_copy(x_hbm.at[idx_v.at[pl.ds(0, _CHUNK)]], rows_a,
                            sem_a).wait()
      writeback(g, rows_a)

      @pl.when(g + 2 < nch)
      def _():
        gather(g + 2, rows_a, sem_a)

      pltpu.make_async_copy(x_hbm.at[idx_v.at[pl.ds(0, _CHUNK)]], rows_b,
                            sem_b).wait()
      writeback(g + 1, rows_b)
      return carry

    lax.fori_loop(0, nch // 2, pair, 0)

  return gather_kernel(x2d, idx_pad)


def _dense_block(xg, mlp_wt, mlp_b, wt, conv_b, nb, k, c, oc):
  xg3 = xg.reshape(nb, k, c)
  rel = (xg3 - xg3[:, 0:1, :]).reshape(nb * k, c)
  logits = (
      jnp.dot(rel, mlp_wt, preferred_element_type=jnp.float32) + mlp_b
  )
  m = jnp.max(logits, axis=1, keepdims=True)
  e = jnp.exp(logits - m)
  p = e / jnp.sum(e, axis=1, keepdims=True)
  p3 = p.reshape(nb, k, k)

  acc = jnp.zeros((nb, k, c), jnp.float32)
  for k1 in range(k):
    acc = acc + p3[:, :, k1][:, :, None] * xg3[:, k1, :][:, None, :]

  out = jnp.broadcast_to(conv_b, (nb, oc))
  for k2 in range(k):
    out = out + jnp.dot(
        acc[:, k2, :], wt[k2], preferred_element_type=jnp.float32
    )
  return jnp.where(out > 0.0, out, jnp.exp(out) - 1.0)


def _tc_dense(xg2d, mlp_wt, mlp_b2, wt, conv_b2, n, nb, k, c, oc):
  nblk = n // nb

  def body(xg_ref, mlpwt_ref, mlpb_ref, wt_ref, convb_ref, out_ref):
    res = _dense_block(
        xg_ref[...], mlpwt_ref[...], mlpb_ref[...], wt_ref[...],
        convb_ref[...], nb, k, c, oc,
    )
    out_ref[...] = res

    @pl.when(pl.program_id(0) == nblk - 1)
    def _zero_last():
      out_ref[nb - 1 : nb, :] = jnp.zeros((1, oc), jnp.float32)

  return pl.pallas_call(
      body,
      grid=(nblk,),
      in_specs=[
          pl.BlockSpec((nb * k, c), lambda i: (i, 0)),
          pl.BlockSpec((c, k), lambda i: (0, 0)),
          pl.BlockSpec((1, k), lambda i: (0, 0)),
          pl.BlockSpec((k, c, oc), lambda i: (0, 0, 0)),
          pl.BlockSpec((1, oc), lambda i: (0, 0)),
      ],
      out_specs=pl.BlockSpec((nb, oc), lambda i: (i, 0)),
      out_shape=jax.ShapeDtypeStruct((n, oc), jnp.float32),
  )(xg2d, mlp_wt, mlp_b2, wt, conv_b2)


def kernel(x, neighbor_index, conv_w, conv_b, mlp_w, mlp_b):
  b, n, c = x.shape
  k = neighbor_index.shape[2]
  oc = conv_w.shape[0]
  assert b == 1

  x2d = x.reshape(n, c)
  ni = neighbor_index.reshape(n * k).astype(jnp.int32)

  rows = n * k
  rows_pad = ((rows + 32 * _CHUNK - 1) // (32 * _CHUNK)) * (32 * _CHUNK)
  ni_pad = jnp.concatenate(
      [ni, jnp.zeros((rows_pad - rows,), jnp.int32)]
  )

  xg2d = _sc_gather(x2d, ni_pad, rows_pad)

  wt = conv_w.reshape(oc, c, k).transpose(2, 1, 0)
  mlp_wt = mlp_w.T
  mlp_b2 = mlp_b.reshape(1, k)
  conv_b2 = conv_b.reshape(1, oc)

  nb = 400
  out2d = _tc_dense(xg2d, mlp_wt, mlp_b2, wt, conv_b2, n, nb, k, c, oc)
  return out2d.reshape(b, n, oc)
